# Initial kernel scaffold; baseline (speedup 1.0000x reference)
import jax
import jax.numpy as jnp
from jax.experimental import pallas as pl


def kernel(x, edge_index, W_self1, W_neigh1, b1, W_self2, W_neigh2, b2):
    raise NotImplementedError("write your pallas kernel here")


# speedup vs baseline: 5.2879x; 5.2879x over previous
import functools

import jax
import jax.numpy as jnp
from jax import lax
from jax.experimental import pallas as pl
from jax.experimental.pallas import tpu as pltpu
from jax.experimental.pallas import tpu_sc as plsc

N_NODES = 10000
N_EDGES = 320000
D_FEAT = 128
D_HID = 128
N_CLASSES = 40

NC = 2
NS = 16
NW = NC * NS
CB = 128
CHUNKS = 79
EPW = CHUNKS * CB
E_PAD = NW * EPW
NROW = 10112
RPT = NROW // NS
W1 = D_HID + 16
W2 = 48

_mesh = plsc.VectorSubcoreMesh(core_axis_name="c", subcore_axis_name="s")


def _make_segsum(width):
    @functools.partial(
        pl.kernel,
        mesh=_mesh,
        compiler_params=pltpu.CompilerParams(use_tc_tiling_on_sc=False),
        out_type=jax.ShapeDtypeStruct((NC, NROW, width), jnp.float32),
        scratch_types=[
            pltpu.VMEM((CHUNKS, CB), jnp.int32),
            pltpu.VMEM((CHUNKS, CB), jnp.int32),
            pltpu.VMEM((CB, width), jnp.float32),
            pltpu.VMEM_SHARED((NROW, width), jnp.float32),
            pltpu.SemaphoreType.DMA,
        ],
    )
    def segsum(table, srci, dsti, zeros_hbm, out, src_v, dst_v, buf, acc, sem):
        c = lax.axis_index("c")
        s = lax.axis_index("s")
        wid = c * NS + s
        r0 = s * RPT
        pltpu.sync_copy(zeros_hbm.at[pl.ds(r0, RPT)], acc.at[pl.ds(r0, RPT)])
        pltpu.sync_copy(srci.at[wid], src_v)
        pltpu.sync_copy(dsti.at[wid], dst_v)
        plsc.subcore_barrier()

        def body(j, carry):
            pltpu.async_copy(table.at[src_v.at[j]], buf, sem).wait()
            pltpu.sync_copy(buf, acc.at[dst_v.at[j]], add=True)
            return carry

        lax.fori_loop(0, CHUNKS, body, 0)
        plsc.subcore_barrier()
        pltpu.sync_copy(acc.at[pl.ds(r0, RPT)], out.at[c, pl.ds(r0, RPT)])

    return segsum


_segsum_w1 = _make_segsum(W1)
_segsum_w2 = _make_segsum(W2)


def _k1(x_ref, w_ref, a_ref, t_ref):
    p = jnp.dot(x_ref[...], w_ref[...], preferred_element_type=jnp.float32)
    a_ref[...] = p[:, :D_FEAT]
    t_ref[:, :D_HID] = p[:, D_FEAT:]
    col = lax.broadcasted_iota(jnp.int32, (x_ref.shape[0], 16), 1)
    t_ref[:, D_HID:] = jnp.where(col == 0, 1.0, 0.0)


def _k2(a1_ref, s_ref, b1_ref, ws2_ref, wn2_ref, b2_ref, oa_ref, og_ref, rc_ref):
    ssum = s_ref[0] + s_ref[1]
    rcnt = 1.0 / jnp.maximum(ssum[:, D_HID : D_HID + 1], 1.0)
    h = jnp.maximum(a1_ref[...] + ssum[:, :D_HID] * rcnt + b1_ref[0:1, :], 0.0)
    oa_ref[...] = (
        jnp.dot(h, ws2_ref[...], preferred_element_type=jnp.float32) + b2_ref[0:1, :]
    )
    og_ref[...] = jnp.dot(h, wn2_ref[...], preferred_element_type=jnp.float32)
    rc_ref[...] = jnp.broadcast_to(rcnt, (rcnt.shape[0], 8))


def _k3(a2_ref, s2_ref, rc_ref, o_ref):
    m2 = (s2_ref[0] + s2_ref[1])[:, :N_CLASSES] * rc_ref[:, 0:1]
    o_ref[...] = a2_ref[...] + m2


def kernel(x, edge_index, W_self1, W_neigh1, b1, W_self2, W_neigh2, b2):
    RB = 1000
    G = N_NODES // RB

    pad = E_PAD - N_EDGES
    src = jnp.concatenate([edge_index[0], jnp.zeros((pad,), jnp.int32)])
    dst = jnp.concatenate(
        [edge_index[1], N_NODES + (jnp.arange(pad, dtype=jnp.int32) % 16)]
    )
    src3 = src.reshape(NW, CHUNKS, CB)
    dst3 = dst.reshape(NW, CHUNKS, CB)
    z1 = jnp.zeros((NROW, W1), jnp.float32)
    z2 = jnp.zeros((NROW, W2), jnp.float32)

    Wcat1 = jnp.concatenate([W_self1, W_neigh1], axis=1)
    A1, T1 = pl.pallas_call(
        _k1,
        grid=(G,),
        in_specs=[
            pl.BlockSpec((RB, D_FEAT), lambda i: (i, 0)),
            pl.BlockSpec((D_FEAT, 2 * D_HID), lambda i: (0, 0)),
        ],
        out_specs=[
            pl.BlockSpec((RB, D_HID), lambda i: (i, 0)),
            pl.BlockSpec((RB, W1), lambda i: (i, 0)),
        ],
        out_shape=[
            jax.ShapeDtypeStruct((N_NODES, D_HID), jnp.float32),
            jax.ShapeDtypeStruct((N_NODES, W1), jnp.float32),
        ],
    )(x, Wcat1)

    S1 = _segsum_w1(T1, src3, dst3, z1)

    b1r = jnp.broadcast_to(b1[None, :], (8, D_HID))
    b2r = jnp.broadcast_to(b2[None, :], (8, N_CLASSES))
    Wn2pad = jnp.concatenate(
        [W_neigh2, jnp.zeros((D_HID, W2 - N_CLASSES), jnp.float32)], axis=1
    )
    A2b, T2, RC = pl.pallas_call(
        _k2,
        grid=(G,),
        in_specs=[
            pl.BlockSpec((RB, D_HID), lambda i: (i, 0)),
            pl.BlockSpec((NC, RB, W1), lambda i: (0, i, 0)),
            pl.BlockSpec((8, D_HID), lambda i: (0, 0)),
            pl.BlockSpec((D_HID, N_CLASSES), lambda i: (0, 0)),
            pl.BlockSpec((D_HID, W2), lambda i: (0, 0)),
            pl.BlockSpec((8, N_CLASSES), lambda i: (0, 0)),
        ],
        out_specs=[
            pl.BlockSpec((RB, N_CLASSES), lambda i: (i, 0)),
            pl.BlockSpec((RB, W2), lambda i: (i, 0)),
            pl.BlockSpec((RB, 8), lambda i: (i, 0)),
        ],
        out_shape=[
            jax.ShapeDtypeStruct((N_NODES, N_CLASSES), jnp.float32),
            jax.ShapeDtypeStruct((N_NODES, W2), jnp.float32),
            jax.ShapeDtypeStruct((N_NODES, 8), jnp.float32),
        ],
    )(A1, S1, b1r, W_self2, Wn2pad, b2r)

    S2 = _segsum_w2(T2, src3, dst3, z2)

    out = pl.pallas_call(
        _k3,
        grid=(G,),
        in_specs=[
            pl.BlockSpec((RB, N_CLASSES), lambda i: (i, 0)),
            pl.BlockSpec((NC, RB, W2), lambda i: (0, i, 0)),
            pl.BlockSpec((RB, 8), lambda i: (i, 0)),
        ],
        out_specs=pl.BlockSpec((RB, N_CLASSES), lambda i: (i, 0)),
        out_shape=jax.ShapeDtypeStruct((N_NODES, N_CLASSES), jnp.float32),
    )(A2b, S2, RC)
    return out


# speedup vs baseline: 6.1566x; 1.1643x over previous
import functools

import jax
import jax.numpy as jnp
from jax import lax
from jax.experimental import pallas as pl
from jax.experimental.pallas import tpu as pltpu
from jax.experimental.pallas import tpu_sc as plsc

N_NODES = 10000
N_EDGES = 320000
D_FEAT = 128
D_HID = 128
N_CLASSES = 40

NC = 2
NS = 16
NW = NC * NS
CB = 64
CHUNKS = 158
EPW = CHUNKS * CB
E_PAD = NW * EPW
NROW = 10112
RPT = NROW // NS
W1 = D_HID + 16
W2 = 48

_mesh = plsc.VectorSubcoreMesh(core_axis_name="c", subcore_axis_name="s")


def _make_segsum(width):
    @functools.partial(
        pl.kernel,
        mesh=_mesh,
        compiler_params=pltpu.CompilerParams(use_tc_tiling_on_sc=False),
        out_type=jax.ShapeDtypeStruct((NC, NROW, width), jnp.float32),
        scratch_types=[
            pltpu.VMEM((CHUNKS, CB), jnp.int32),
            pltpu.VMEM((CHUNKS, CB), jnp.int32),
            pltpu.VMEM((CB, width), jnp.float32),
            pltpu.VMEM((CB, width), jnp.float32),
            pltpu.VMEM_SHARED((NROW, width), jnp.float32),
            pltpu.SemaphoreType.DMA,
            pltpu.SemaphoreType.DMA,
        ],
    )
    def segsum(table, srci, dsti, zeros_hbm, out, src_v, dst_v, bufa, bufb, acc,
               sema, semb):
        c = lax.axis_index("c")
        s = lax.axis_index("s")
        wid = c * NS + s
        r0 = s * RPT
        pltpu.sync_copy(zeros_hbm.at[pl.ds(r0, RPT)], acc.at[pl.ds(r0, RPT)])
        pltpu.sync_copy(srci.at[wid], src_v)
        pltpu.sync_copy(dsti.at[wid], dst_v)
        plsc.subcore_barrier()

        pltpu.async_copy(table.at[src_v.at[0]], bufa, sema)

        def body(i, carry):
            j0 = 2 * i
            pltpu.async_copy(table.at[src_v.at[j0 + 1]], bufb, semb)
            pltpu.make_async_copy(table.at[src_v.at[j0]], bufa, sema).wait()
            pltpu.sync_copy(bufa, acc.at[dst_v.at[j0]], add=True)
            pltpu.async_copy(table.at[src_v.at[j0 + 2]], bufa, sema)
            pltpu.make_async_copy(table.at[src_v.at[j0 + 1]], bufb, semb).wait()
            pltpu.sync_copy(bufb, acc.at[dst_v.at[j0 + 1]], add=True)
            return carry

        lax.fori_loop(0, (CHUNKS - 2) // 2, body, 0)
        pltpu.async_copy(table.at[src_v.at[CHUNKS - 1]], bufb, semb)
        pltpu.make_async_copy(table.at[src_v.at[CHUNKS - 2]], bufa, sema).wait()
        pltpu.sync_copy(bufa, acc.at[dst_v.at[CHUNKS - 2]], add=True)
        pltpu.make_async_copy(table.at[src_v.at[CHUNKS - 1]], bufb, semb).wait()
        pltpu.sync_copy(bufb, acc.at[dst_v.at[CHUNKS - 1]], add=True)
        plsc.subcore_barrier()
        pltpu.sync_copy(acc.at[pl.ds(r0, RPT)], out.at[c, pl.ds(r0, RPT)])

    return segsum


_segsum_w1 = _make_segsum(W1)
_segsum_w2 = _make_segsum(W2)


def _k1(x_ref, w_ref, a_ref, t_ref):
    p = jnp.dot(x_ref[...], w_ref[...], preferred_element_type=jnp.float32)
    a_ref[...] = p[:, :D_FEAT]
    t_ref[:, :D_HID] = p[:, D_FEAT:]
    col = lax.broadcasted_iota(jnp.int32, (x_ref.shape[0], 16), 1)
    t_ref[:, D_HID:] = jnp.where(col == 0, 1.0, 0.0)


def _k2(a1_ref, s_ref, b1_ref, ws2_ref, wn2_ref, b2_ref, oa_ref, og_ref, rc_ref):
    ssum = s_ref[0] + s_ref[1]
    rcnt = 1.0 / jnp.maximum(ssum[:, D_HID : D_HID + 1], 1.0)
    h = jnp.maximum(a1_ref[...] + ssum[:, :D_HID] * rcnt + b1_ref[0:1, :], 0.0)
    oa_ref[...] = (
        jnp.dot(h, ws2_ref[...], preferred_element_type=jnp.float32) + b2_ref[0:1, :]
    )
    og_ref[...] = jnp.dot(h, wn2_ref[...], preferred_element_type=jnp.float32)
    rc_ref[...] = jnp.broadcast_to(rcnt, (rcnt.shape[0], 8))


def _k3(a2_ref, s2_ref, rc_ref, o_ref):
    m2 = (s2_ref[0] + s2_ref[1])[:, :N_CLASSES] * rc_ref[:, 0:1]
    o_ref[...] = a2_ref[...] + m2


def kernel(x, edge_index, W_self1, W_neigh1, b1, W_self2, W_neigh2, b2):
    RB = 1000
    G = N_NODES // RB

    pad = E_PAD - N_EDGES
    src = jnp.concatenate([edge_index[0], jnp.zeros((pad,), jnp.int32)])
    dst = jnp.concatenate(
        [edge_index[1], N_NODES + (jnp.arange(pad, dtype=jnp.int32) % 16)]
    )
    src3 = src.reshape(NW, CHUNKS, CB)
    dst3 = dst.reshape(NW, CHUNKS, CB)
    z1 = jnp.zeros((NROW, W1), jnp.float32)
    z2 = jnp.zeros((NROW, W2), jnp.float32)

    Wcat1 = jnp.concatenate([W_self1, W_neigh1], axis=1)
    A1, T1 = pl.pallas_call(
        _k1,
        grid=(G,),
        in_specs=[
            pl.BlockSpec((RB, D_FEAT), lambda i: (i, 0)),
            pl.BlockSpec((D_FEAT, 2 * D_HID), lambda i: (0, 0)),
        ],
        out_specs=[
            pl.BlockSpec((RB, D_HID), lambda i: (i, 0)),
            pl.BlockSpec((RB, W1), lambda i: (i, 0)),
        ],
        out_shape=[
            jax.ShapeDtypeStruct((N_NODES, D_HID), jnp.float32),
            jax.ShapeDtypeStruct((N_NODES, W1), jnp.float32),
        ],
    )(x, Wcat1)

    S1 = _segsum_w1(T1, src3, dst3, z1)

    b1r = jnp.broadcast_to(b1[None, :], (8, D_HID))
    b2r = jnp.broadcast_to(b2[None, :], (8, N_CLASSES))
    Wn2pad = jnp.concatenate(
        [W_neigh2, jnp.zeros((D_HID, W2 - N_CLASSES), jnp.float32)], axis=1
    )
    A2b, T2, RC = pl.pallas_call(
        _k2,
        grid=(G,),
        in_specs=[
            pl.BlockSpec((RB, D_HID), lambda i: (i, 0)),
            pl.BlockSpec((NC, RB, W1), lambda i: (0, i, 0)),
            pl.BlockSpec((8, D_HID), lambda i: (0, 0)),
            pl.BlockSpec((D_HID, N_CLASSES), lambda i: (0, 0)),
            pl.BlockSpec((D_HID, W2), lambda i: (0, 0)),
            pl.BlockSpec((8, N_CLASSES), lambda i: (0, 0)),
        ],
        out_specs=[
            pl.BlockSpec((RB, N_CLASSES), lambda i: (i, 0)),
            pl.BlockSpec((RB, W2), lambda i: (i, 0)),
            pl.BlockSpec((RB, 8), lambda i: (i, 0)),
        ],
        out_shape=[
            jax.ShapeDtypeStruct((N_NODES, N_CLASSES), jnp.float32),
            jax.ShapeDtypeStruct((N_NODES, W2), jnp.float32),
            jax.ShapeDtypeStruct((N_NODES, 8), jnp.float32),
        ],
    )(A1, S1, b1r, W_self2, Wn2pad, b2r)

    S2 = _segsum_w2(T2, src3, dst3, z2)

    out = pl.pallas_call(
        _k3,
        grid=(G,),
        in_specs=[
            pl.BlockSpec((RB, N_CLASSES), lambda i: (i, 0)),
            pl.BlockSpec((NC, RB, W2), lambda i: (0, i, 0)),
            pl.BlockSpec((RB, 8), lambda i: (i, 0)),
        ],
        out_specs=pl.BlockSpec((RB, N_CLASSES), lambda i: (i, 0)),
        out_shape=jax.ShapeDtypeStruct((N_NODES, N_CLASSES), jnp.float32),
    )(A2b, S2, RC)
    return out
